# Initial kernel scaffold; baseline (speedup 1.0000x reference)
import jax
import jax.numpy as jnp
from jax.experimental import pallas as pl


def kernel(embedded_words, action_vectors):
    raise NotImplementedError("write your pallas kernel here")


# speedup vs baseline: 14.5573x; 14.5573x over previous
import jax
import jax.numpy as jnp
from jax.experimental import pallas as pl

ACTION_SIZE = 4
POINT_SIZE = 11
EMBED_DIM = 128
BLK = 2048


def _decoder_body(x_ref, av_ref, out_ref):
    e = x_ref[:]
    av = av_ref[:]
    num = jax.lax.dot_general(
        e, av, (((1,), (0,)), ((), ())),
        preferred_element_type=jnp.float32,
        precision=jax.lax.Precision.HIGHEST)
    n1 = jnp.sqrt(jnp.sum(e * e, axis=1, keepdims=True))
    n2 = jnp.sqrt(jnp.sum(av * av, axis=0, keepdims=True))
    sims = num / jnp.maximum(n1 * n2, 1e-8)
    g0 = jnp.max(sims[:, 0:4], axis=1, keepdims=True)
    g1 = jnp.max(sims[:, 4:9], axis=1, keepdims=True)
    g2 = sims[:, 9:10]
    g3 = sims[:, 10:11]
    mx = jnp.maximum(jnp.maximum(g0, g1), jnp.maximum(g2, g3))
    idx = jnp.where(g0 >= mx, 0,
                    jnp.where(g1 >= mx, 1,
                              jnp.where(g2 >= mx, 2, 3)))
    lanes = jax.lax.broadcasted_iota(jnp.int32, (x_ref.shape[0], ACTION_SIZE), 1)
    out_ref[:] = (lanes == idx).astype(jnp.float32)


def kernel(embedded_words, action_vectors):
    batch = embedded_words.shape[0]
    av = action_vectors[0]
    av_pad = jnp.pad(av, ((0, 0), (0, 16 - POINT_SIZE)))
    grid = (batch // BLK,)
    return pl.pallas_call(
        _decoder_body,
        grid=grid,
        in_specs=[
            pl.BlockSpec((BLK, EMBED_DIM), lambda i: (i, 0)),
            pl.BlockSpec((EMBED_DIM, 16), lambda i: (0, 0)),
        ],
        out_specs=pl.BlockSpec((BLK, ACTION_SIZE), lambda i: (i, 0)),
        out_shape=jax.ShapeDtypeStruct((batch, ACTION_SIZE), jnp.float32),
    )(embedded_words, av_pad)
